# baseline (speedup 1.0000x reference)
import functools
import math

import jax
import jax.numpy as jnp
from jax import lax
from jax.experimental import pallas as pl
from jax.experimental.pallas import tpu as pltpu
from jax.experimental.pallas import tpu_sc as plsc

N, F, E = 10000, 256, 160000
M_TILE = 400
C = 128
NBC = E // C
NC, NS, L = 2, 16, 16
NW = NC * NS
NB_LO = 38
NHVY = (NBC - NW * NB_LO) // 2
E_LO = NB_LO * C
E_HI = (NB_LO + 2) * C


def _pack_rows_i32(y):
    h = F // 2
    zi = lax.bitcast_convert_type(y.astype(jnp.bfloat16), jnp.int16)
    lo = zi[:, :h].astype(jnp.int32) & 0xFFFF
    hi = zi[:, h:].astype(jnp.int32) << 16
    return hi | lo


def _proj_kernel(x_ref, w_ref, b_ref, q_ref, k_ref, *, scale):
    res = jnp.dot(x_ref[...], w_ref[...], preferred_element_type=jnp.float32)
    res = res + b_ref[...]
    q_ref[...] = _pack_rows_i32(res[:, :F] * scale)
    k_ref[...] = _pack_rows_i32(res[:, F:])


def _project(x, W, b, scale):
    return pl.pallas_call(
        functools.partial(_proj_kernel, scale=scale),
        grid=(N // M_TILE,),
        in_specs=[
            pl.BlockSpec((M_TILE, F), lambda i: (i, 0)),
            pl.BlockSpec((F, 2 * F), lambda i: (0, 0)),
            pl.BlockSpec((1, 2 * F), lambda i: (0, 0)),
        ],
        out_specs=[
            pl.BlockSpec((M_TILE, F // 2), lambda i: (i, 0)),
            pl.BlockSpec((M_TILE, F // 2), lambda i: (i, 0)),
        ],
        out_shape=[
            jax.ShapeDtypeStruct((N, F // 2), jnp.int32),
            jax.ShapeDtypeStruct((N, F // 2), jnp.int32),
        ],
    )(x, W, b)


def _sc_edge_dot(q, k, idx_i, idx_j):
    mesh = plsc.VectorSubcoreMesh(core_axis_name="c", subcore_axis_name="s")

    @functools.partial(
        pl.kernel,
        mesh=mesh,
        out_type=jax.ShapeDtypeStruct((E,), jnp.float32),
        scratch_types=[
            pltpu.VMEM((E_HI,), jnp.int32),
            pltpu.VMEM((E_HI,), jnp.int32),
            [pltpu.VMEM((C, F // 2), jnp.int32)] * 2,
            [pltpu.VMEM((C, F // 2), jnp.int32)] * 2,
            pltpu.VMEM((E_HI,), jnp.float32),
            pltpu.VMEM((L * L,), jnp.float32),
            [pltpu.SemaphoreType.DMA] * 2,
        ],
        compiler_params=pltpu.CompilerParams(needs_layout_passes=False),
    )
    def sc_kernel(q_hbm, k_hbm, ii_hbm, jj_hbm, out_hbm,
                  ii_v, jj_v, qrs, krs, out_v, accflat, sems):
        wid = lax.axis_index("s") * NC + lax.axis_index("c")
        hvy = jnp.minimum(wid, NHVY)
        nb = jnp.where(wid < NHVY, NB_LO + 2, NB_LO)
        ebase = (NB_LO * wid + 2 * hvy) * C

        pltpu.sync_copy(ii_hbm.at[pl.ds(ebase, E_LO)], ii_v.at[pl.ds(0, E_LO)])
        pltpu.sync_copy(jj_hbm.at[pl.ds(ebase, E_LO)], jj_v.at[pl.ds(0, E_LO)])

        @pl.when(wid < NHVY)
        def _():
            pltpu.sync_copy(ii_hbm.at[pl.ds(ebase + E_LO, E_HI - E_LO)],
                            ii_v.at[pl.ds(E_LO, E_HI - E_LO)])
            pltpu.sync_copy(jj_hbm.at[pl.ds(ebase + E_LO, E_HI - E_LO)],
                            jj_v.at[pl.ds(E_LO, E_HI - E_LO)])

        def issue(blk, qr, kr, sem):
            pltpu.async_copy(q_hbm.at[ii_v.at[pl.ds(blk * C, C)]], qr, sem)
            pltpu.async_copy(k_hbm.at[jj_v.at[pl.ds(blk * C, C)]], kr, sem)

        def drain(qr, kr, sem):
            pltpu.make_async_copy(q_hbm.at[pl.ds(0, C)], qr, sem).wait()
            pltpu.make_async_copy(k_hbm.at[pl.ds(0, C)], kr, sem).wait()

        lane = lax.iota(jnp.int32, L)

        def compute(blk, qr, kr):
            def group_body(g, c2):
                for p in range(L):
                    acc = None
                    for s in range(F // (4 * L)):
                        qv0 = plsc.bitcast(
                            qr[g * L + p, pl.ds(2 * s * L, L)], jnp.bfloat16)
                        kv0 = plsc.bitcast(
                            kr[g * L + p, pl.ds(2 * s * L, L)], jnp.bfloat16)
                        qv1 = plsc.bitcast(
                            qr[g * L + p, pl.ds((2 * s + 1) * L, L)],
                            jnp.bfloat16)
                        kv1 = plsc.bitcast(
                            kr[g * L + p, pl.ds((2 * s + 1) * L, L)],
                            jnp.bfloat16)
                        pa, pb = plsc.unpack(
                            qv0 * kv0 + qv1 * kv1,
                            format=plsc.PackFormat.INTERLEAVED)
                        term = pa + pb
                        acc = term if acc is None else acc + term
                    accflat[pl.ds(p * L, L)] = acc
                outvec = plsc.load_gather(accflat, [lane * L])
                for c in range(1, L):
                    outvec = outvec + plsc.load_gather(accflat, [lane * L + c])
                out_v[pl.ds(blk * C + g * L, L)] = outvec
                return c2

            lax.fori_loop(0, C // L, group_body, 0)

        issue(0, qrs[0], krs[0], sems[0])

        def pair_body(i, carry):
            b0 = 2 * i
            issue(b0 + 1, qrs[1], krs[1], sems[1])
            drain(qrs[0], krs[0], sems[0])
            compute(b0, qrs[0], krs[0])

            @pl.when(b0 + 2 < nb)
            def _():
                issue(b0 + 2, qrs[0], krs[0], sems[0])

            drain(qrs[1], krs[1], sems[1])
            compute(b0 + 1, qrs[1], krs[1])
            return carry

        lax.fori_loop(0, nb // 2, pair_body, 0)

        pltpu.sync_copy(out_v.at[pl.ds(0, E_LO)], out_hbm.at[pl.ds(ebase, E_LO)])

        @pl.when(wid < NHVY)
        def _():
            pltpu.sync_copy(out_v.at[pl.ds(E_LO, E_HI - E_LO)],
                            out_hbm.at[pl.ds(ebase + E_LO, E_HI - E_LO)])

    return sc_kernel(q, k, idx_i, idx_j)


def kernel(x, idx_i, idx_j, Wq, bq, Wk, bk):
    scale = 1.0 / math.sqrt(F)
    W = jnp.concatenate([Wq, Wk], axis=1)
    b = jnp.concatenate([bq, bk])[None, :]
    q32, k32 = _project(x, W, b, scale)
    return _sc_edge_dot(q32, k32,
                        idx_i.astype(jnp.int32), idx_j.astype(jnp.int32))


# speedup vs baseline: 1.0197x; 1.0197x over previous
import functools
import math

import jax
import jax.numpy as jnp
from jax import lax
from jax.experimental import pallas as pl
from jax.experimental.pallas import tpu as pltpu
from jax.experimental.pallas import tpu_sc as plsc

N, F, E = 10000, 256, 160000
M_TILE = 400
C = 128
NBC = E // C
NC, NS, L = 2, 16, 16
NW = NC * NS
NB_LO = 38
NHVY = (NBC - NW * NB_LO) // 2
E_LO = NB_LO * C
E_HI = (NB_LO + 2) * C


def _pack_rows_i32(y):
    h = F // 2
    zi = lax.bitcast_convert_type(y.astype(jnp.bfloat16), jnp.int16)
    lo = zi[:, :h].astype(jnp.int32) & 0xFFFF
    hi = zi[:, h:].astype(jnp.int32) << 16
    return hi | lo


def _proj_kernel(x_ref, w_ref, b_ref, q_ref, k_ref, *, scale):
    res = jnp.dot(x_ref[...].astype(jnp.bfloat16), w_ref[...],
                  preferred_element_type=jnp.float32)
    res = res + b_ref[...]
    q_ref[...] = _pack_rows_i32(res[:, :F] * scale)
    k_ref[...] = _pack_rows_i32(res[:, F:])


def _project(x, W, b, scale):
    return pl.pallas_call(
        functools.partial(_proj_kernel, scale=scale),
        grid=(N // M_TILE,),
        in_specs=[
            pl.BlockSpec((M_TILE, F), lambda i: (i, 0)),
            pl.BlockSpec((F, 2 * F), lambda i: (0, 0)),
            pl.BlockSpec((1, 2 * F), lambda i: (0, 0)),
        ],
        out_specs=[
            pl.BlockSpec((M_TILE, F // 2), lambda i: (i, 0)),
            pl.BlockSpec((M_TILE, F // 2), lambda i: (i, 0)),
        ],
        out_shape=[
            jax.ShapeDtypeStruct((N, F // 2), jnp.int32),
            jax.ShapeDtypeStruct((N, F // 2), jnp.int32),
        ],
    )(x, W, b)


def _sc_edge_dot(q, k, idx_i, idx_j):
    mesh = plsc.VectorSubcoreMesh(core_axis_name="c", subcore_axis_name="s")

    @functools.partial(
        pl.kernel,
        mesh=mesh,
        out_type=jax.ShapeDtypeStruct((E,), jnp.float32),
        scratch_types=[
            pltpu.VMEM((E_HI,), jnp.int32),
            pltpu.VMEM((E_HI,), jnp.int32),
            [pltpu.VMEM((C, F // 2), jnp.int32)] * 2,
            [pltpu.VMEM((C, F // 2), jnp.int32)] * 2,
            pltpu.VMEM((E_HI,), jnp.float32),
            pltpu.VMEM((L * L,), jnp.float32),
            [pltpu.SemaphoreType.DMA] * 2,
        ],
        compiler_params=pltpu.CompilerParams(needs_layout_passes=False),
    )
    def sc_kernel(q_hbm, k_hbm, ii_hbm, jj_hbm, out_hbm,
                  ii_v, jj_v, qrs, krs, out_v, accflat, sems):
        wid = lax.axis_index("s") * NC + lax.axis_index("c")
        hvy = jnp.minimum(wid, NHVY)
        nb = jnp.where(wid < NHVY, NB_LO + 2, NB_LO)
        ebase = (NB_LO * wid + 2 * hvy) * C

        pltpu.sync_copy(ii_hbm.at[pl.ds(ebase, E_LO)], ii_v.at[pl.ds(0, E_LO)])
        pltpu.sync_copy(jj_hbm.at[pl.ds(ebase, E_LO)], jj_v.at[pl.ds(0, E_LO)])

        @pl.when(wid < NHVY)
        def _():
            pltpu.sync_copy(ii_hbm.at[pl.ds(ebase + E_LO, E_HI - E_LO)],
                            ii_v.at[pl.ds(E_LO, E_HI - E_LO)])
            pltpu.sync_copy(jj_hbm.at[pl.ds(ebase + E_LO, E_HI - E_LO)],
                            jj_v.at[pl.ds(E_LO, E_HI - E_LO)])

        def issue(blk, qr, kr, sem):
            pltpu.async_copy(q_hbm.at[ii_v.at[pl.ds(blk * C, C)]], qr, sem)
            pltpu.async_copy(k_hbm.at[jj_v.at[pl.ds(blk * C, C)]], kr, sem)

        def drain(qr, kr, sem):
            pltpu.make_async_copy(q_hbm.at[pl.ds(0, C)], qr, sem).wait()
            pltpu.make_async_copy(k_hbm.at[pl.ds(0, C)], kr, sem).wait()

        lane = lax.iota(jnp.int32, L)

        def compute(blk, qr, kr):
            def group_body(g, c2):
                for p in range(L):
                    acc = None
                    for s in range(F // (2 * L)):
                        qv = plsc.bitcast(qr[g * L + p, pl.ds(s * L, L)],
                                          jnp.bfloat16)
                        kv = plsc.bitcast(kr[g * L + p, pl.ds(s * L, L)],
                                          jnp.bfloat16)
                        pa, pb = plsc.unpack(
                            qv * kv, format=plsc.PackFormat.INTERLEAVED)
                        term = pa + pb
                        acc = term if acc is None else acc + term
                    accflat[pl.ds(p * L, L)] = acc
                outvec = plsc.load_gather(accflat, [lane * L])
                for c in range(1, L):
                    outvec = outvec + plsc.load_gather(accflat, [lane * L + c])
                out_v[pl.ds(blk * C + g * L, L)] = outvec
                return c2

            lax.fori_loop(0, C // L, group_body, 0)

        issue(0, qrs[0], krs[0], sems[0])

        def pair_body(i, carry):
            b0 = 2 * i
            issue(b0 + 1, qrs[1], krs[1], sems[1])
            drain(qrs[0], krs[0], sems[0])
            compute(b0, qrs[0], krs[0])

            @pl.when(b0 + 2 < nb)
            def _():
                issue(b0 + 2, qrs[0], krs[0], sems[0])

            drain(qrs[1], krs[1], sems[1])
            compute(b0 + 1, qrs[1], krs[1])
            return carry

        lax.fori_loop(0, nb // 2, pair_body, 0)

        pltpu.sync_copy(out_v.at[pl.ds(0, E_LO)], out_hbm.at[pl.ds(ebase, E_LO)])

        @pl.when(wid < NHVY)
        def _():
            pltpu.sync_copy(out_v.at[pl.ds(E_LO, E_HI - E_LO)],
                            out_hbm.at[pl.ds(ebase + E_LO, E_HI - E_LO)])

    return sc_kernel(q, k, idx_i, idx_j)


def kernel(x, idx_i, idx_j, Wq, bq, Wk, bk):
    scale = 1.0 / math.sqrt(F)
    W = jnp.concatenate([Wq, Wk], axis=1).astype(jnp.bfloat16)
    b = jnp.concatenate([bq, bk])[None, :]
    q32, k32 = _project(x, W, b, scale)
    return _sc_edge_dot(q32, k32,
                        idx_i.astype(jnp.int32), idx_j.astype(jnp.int32))


# speedup vs baseline: 1.0232x; 1.0035x over previous
import functools
import math

import jax
import jax.numpy as jnp
from jax import lax
from jax.experimental import pallas as pl
from jax.experimental.pallas import tpu as pltpu
from jax.experimental.pallas import tpu_sc as plsc

N, F, E = 10000, 256, 160000
M_TILE = 400
C = 128
NBC = E // C
NC, NS, L = 2, 16, 16
NW = NC * NS
NB_LO = 38
NHVY = (NBC - NW * NB_LO) // 2
E_LO = NB_LO * C
E_HI = (NB_LO + 2) * C


def _pack_rows_i32(y):
    h = F // 2
    zi = lax.bitcast_convert_type(y.astype(jnp.bfloat16), jnp.int16)
    lo = zi[:, :h].astype(jnp.int32) & 0xFFFF
    hi = zi[:, h:].astype(jnp.int32) << 16
    return hi | lo


def _proj_kernel(x_ref, w_ref, b_ref, q_ref, k_ref, *, scale):
    res = jnp.dot(x_ref[...], w_ref[...], preferred_element_type=jnp.float32)
    res = res + b_ref[...]
    q_ref[...] = _pack_rows_i32(res[:, :F] * scale)
    k_ref[...] = _pack_rows_i32(res[:, F:])


def _project(x, W, b, scale):
    return pl.pallas_call(
        functools.partial(_proj_kernel, scale=scale),
        grid=(N // M_TILE,),
        in_specs=[
            pl.BlockSpec((M_TILE, F), lambda i: (i, 0)),
            pl.BlockSpec((F, 2 * F), lambda i: (0, 0)),
            pl.BlockSpec((1, 2 * F), lambda i: (0, 0)),
        ],
        out_specs=[
            pl.BlockSpec((M_TILE, F // 2), lambda i: (i, 0)),
            pl.BlockSpec((M_TILE, F // 2), lambda i: (i, 0)),
        ],
        out_shape=[
            jax.ShapeDtypeStruct((N, F // 2), jnp.int32),
            jax.ShapeDtypeStruct((N, F // 2), jnp.int32),
        ],
    )(x, W, b)


def _sc_edge_dot(q, k, idx_i, idx_j):
    mesh = plsc.VectorSubcoreMesh(core_axis_name="c", subcore_axis_name="s")

    @functools.partial(
        pl.kernel,
        mesh=mesh,
        out_type=jax.ShapeDtypeStruct((E,), jnp.float32),
        scratch_types=[
            pltpu.VMEM((E_HI,), jnp.int32),
            pltpu.VMEM((E_HI,), jnp.int32),
            [pltpu.VMEM((C, F // 2), jnp.int32)] * 2,
            [pltpu.VMEM((C, F // 2), jnp.int32)] * 2,
            pltpu.VMEM((E_HI,), jnp.float32),
            pltpu.VMEM((L * L,), jnp.float32),
            [pltpu.SemaphoreType.DMA] * 2,
        ],
        compiler_params=pltpu.CompilerParams(needs_layout_passes=False),
    )
    def sc_kernel(q_hbm, k_hbm, ii_hbm, jj_hbm, out_hbm,
                  ii_v, jj_v, qrs, krs, out_v, accflat, sems):
        wid = lax.axis_index("s") * NC + lax.axis_index("c")
        hvy = jnp.minimum(wid, NHVY)
        nb = jnp.where(wid < NHVY, NB_LO + 2, NB_LO)
        ebase = (NB_LO * wid + 2 * hvy) * C

        pltpu.sync_copy(ii_hbm.at[pl.ds(ebase, E_LO)], ii_v.at[pl.ds(0, E_LO)])
        pltpu.sync_copy(jj_hbm.at[pl.ds(ebase, E_LO)], jj_v.at[pl.ds(0, E_LO)])

        @pl.when(wid < NHVY)
        def _():
            pltpu.sync_copy(ii_hbm.at[pl.ds(ebase + E_LO, E_HI - E_LO)],
                            ii_v.at[pl.ds(E_LO, E_HI - E_LO)])
            pltpu.sync_copy(jj_hbm.at[pl.ds(ebase + E_LO, E_HI - E_LO)],
                            jj_v.at[pl.ds(E_LO, E_HI - E_LO)])

        def issue(blk, qr, kr, sem):
            pltpu.async_copy(q_hbm.at[ii_v.at[pl.ds(blk * C, C)]], qr, sem)
            pltpu.async_copy(k_hbm.at[jj_v.at[pl.ds(blk * C, C)]], kr, sem)

        def drain(qr, kr, sem):
            pltpu.make_async_copy(q_hbm.at[pl.ds(0, C)], qr, sem).wait()
            pltpu.make_async_copy(k_hbm.at[pl.ds(0, C)], kr, sem).wait()

        lane = lax.iota(jnp.int32, L)

        def compute(blk, qr, kr):
            def group_body(g, c2):
                for p in range(L):
                    acc = None
                    for s in range(F // (2 * L)):
                        qv = plsc.bitcast(qr[g * L + p, pl.ds(s * L, L)],
                                          jnp.bfloat16)
                        kv = plsc.bitcast(kr[g * L + p, pl.ds(s * L, L)],
                                          jnp.bfloat16)
                        pa, pb = plsc.unpack(
                            qv * kv, format=plsc.PackFormat.INTERLEAVED)
                        term = pa + pb
                        acc = term if acc is None else acc + term
                    accflat[pl.ds(p * L, L)] = acc
                outvec = plsc.load_gather(accflat, [lane * L])
                for c in range(1, L):
                    outvec = outvec + plsc.load_gather(accflat, [lane * L + c])
                out_v[pl.ds(blk * C + g * L, L)] = outvec
                return c2

            lax.fori_loop(0, C // L, group_body, 0)

        issue(0, qrs[0], krs[0], sems[0])

        def pair_body(i, carry):
            b0 = 2 * i
            issue(b0 + 1, qrs[1], krs[1], sems[1])
            drain(qrs[0], krs[0], sems[0])
            compute(b0, qrs[0], krs[0])

            @pl.when(b0 + 2 < nb)
            def _():
                issue(b0 + 2, qrs[0], krs[0], sems[0])

            drain(qrs[1], krs[1], sems[1])
            compute(b0 + 1, qrs[1], krs[1])
            return carry

        lax.fori_loop(0, nb // 2, pair_body, 0)

        pltpu.sync_copy(out_v.at[pl.ds(0, E_LO)], out_hbm.at[pl.ds(ebase, E_LO)])

        @pl.when(wid < NHVY)
        def _():
            pltpu.sync_copy(out_v.at[pl.ds(E_LO, E_HI - E_LO)],
                            out_hbm.at[pl.ds(ebase + E_LO, E_HI - E_LO)])

    return sc_kernel(q, k, idx_i, idx_j)


def kernel(x, idx_i, idx_j, Wq, bq, Wk, bk):
    scale = 1.0 / math.sqrt(F)
    W = jnp.concatenate([Wq, Wk], axis=1)
    b = jnp.concatenate([bq, bk])[None, :]
    q32, k32 = _project(x, W, b, scale)
    return _sc_edge_dot(q32, k32,
                        idx_i.astype(jnp.int32), idx_j.astype(jnp.int32))
